# baseline (speedup 1.0000x reference)
import functools

import jax
import jax.numpy as jnp
from jax import lax
from jax.experimental import pallas as pl
from jax.experimental.pallas import tpu as pltpu
from jax.experimental.pallas import tpu_sc as plsc

_T = 4
_G = 64
_NC = 2
_NS = 16
_NW = _NC * _NS
_CH = 80


@functools.lru_cache(maxsize=None)
def _make_spmm(N, E, D):
    ept = E // _NW
    nch = ept // _CH
    npad = -(-N // 128) * 128
    rpt = npad // _NS
    zfull = rpt // _CH
    ztail = rpt - zfull * _CH
    mesh = plsc.VectorSubcoreMesh(core_axis_name="c", subcore_axis_name="s")

    @functools.partial(
        pl.kernel,
        out_type=jax.ShapeDtypeStruct((_NC, npad, D), jnp.float32),
        mesh=mesh,
        scratch_types=(
            [pltpu.VMEM((2, _CH), jnp.int32) for _ in range(6)]
            + [pltpu.VMEM((_CH, D), jnp.float32) for _ in range(3)]
            + [pltpu.VMEM_SHARED((npad, D), jnp.float32)]
            + [pltpu.SemaphoreType.DMA] * 13
        ),
    )
    def spmm(projs, edge3, out, i0, i1, i2, i3, i4, i5, r0, r1, r2, acc,
             si0, si1, si2, si3, si4, si5, sg0, sg1, sg2, sc0, sc1, sc2, sz):
        c = lax.axis_index("c")
        s = lax.axis_index("s")
        wid = s * _NC + c
        islot = [i0, i1, i2, i3, i4, i5]
        rows = [r0, r1, r2]
        sidx = [si0, si1, si2, si3, si4, si5]
        sg = [sg0, sg1, sg2]
        ssc = [sc0, sc1, sc2]

        nvec = _CH * (D // 16)

        def zfill(i, carry):
            r0[i // (D // 16), pl.ds((i % (D // 16)) * 16, 16)] = (
                jnp.zeros((16,), jnp.float32))
            return carry

        lax.fori_loop(0, nvec, zfill, 0)

        def zcopy(k, carry):
            pltpu.async_copy(r0, acc.at[pl.ds(s * rpt + k * _CH, _CH)], sz)
            return carry

        lax.fori_loop(0, zfull, zcopy, 0)
        if ztail:
            pltpu.async_copy(r0.at[pl.ds(0, ztail)],
                             acc.at[pl.ds(s * rpt + zfull * _CH, ztail)], sz)

        def zwait(k, carry):
            pltpu.make_async_copy(r0, acc.at[pl.ds(0, _CH)], sz).wait()
            return carry

        lax.fori_loop(0, zfull, zwait, 0)
        if ztail:
            pltpu.make_async_copy(r0.at[pl.ds(0, ztail)],
                                  acc.at[pl.ds(0, ztail)], sz).wait()
        plsc.subcore_barrier()

        last = nch - 1

        def fetch_idx(j, q):
            pltpu.async_copy(edge3.at[wid, j], islot[q], sidx[q])

        def wait_idx(q):
            pltpu.make_async_copy(edge3.at[wid, 0], islot[q],
                                  sidx[q]).wait()

        def gather(j, q, g):
            pltpu.async_copy(projs.at[islot[q].at[0]], rows[g], sg[g])

        def wait_gather(b):
            pltpu.make_async_copy(projs.at[islot[0].at[0]], rows[b],
                                  sg[b]).wait()

        def scatter(b, qb):
            pltpu.async_copy(rows[b], acc.at[islot[qb].at[1]], ssc[b],
                             add=True)

        def wait_scatter(b):
            pltpu.make_async_copy(rows[b], acc.at[pl.ds(0, _CH)],
                                  ssc[b]).wait()

        def step(j, jj, k):
            b, g = j % 3, (j + 1) % 3
            q, f, qb = (j + 1) % 6, (j + 4) % 6, j % 6
            jd = j if jj is None else jj + k
            if j >= 2:
                wait_scatter(g)
            if j + 1 <= last:
                wait_idx(q)
                gather(jd + 1, q, g)
            if j + 4 <= last:
                fetch_idx(jd + 4, f)
            wait_gather(b)
            scatter(b, qb)

        for j in range(4):
            fetch_idx(j, j)
        wait_idx(0)
        gather(0, 0, 0)
        for j in range(6):
            step(j, None, None)

        ngrp = (last + 1 - 6 - 5) // 6

        def group(js, carry):
            jj = 6 + 6 * js
            for k in range(6):
                step(6 + k, jj, k)
            return carry

        lax.fori_loop(0, ngrp, group, 0)
        for j in range(6 + 6 * ngrp, last + 1):
            step(j, None, None)
        wait_scatter((last - 1) % 3)
        wait_scatter(last % 3)

        plsc.subcore_barrier()
        pltpu.sync_copy(acc.at[pl.ds(s * rpt, rpt)],
                        out.at[c, pl.ds(s * rpt, rpt)])

    return spmm


def _tc_proj(m, wmt):
    n, d = m.shape
    blk = 1000

    def body(m_ref, w_ref, o_ref):
        o_ref[...] = jnp.dot(m_ref[...], w_ref[...],
                             preferred_element_type=jnp.float32)

    return pl.pallas_call(
        body,
        grid=(n // blk,),
        in_specs=[pl.BlockSpec((blk, d), lambda i: (i, 0)),
                  pl.BlockSpec((d, d), lambda i: (0, 0))],
        out_specs=pl.BlockSpec((blk, d), lambda i: (i, 0)),
        out_shape=jax.ShapeDtypeStruct((n, d), jnp.float32),
    )(m, wmt)


def _tc_gh(m, whht, bhh):
    n, d = m.shape
    blk = 1000

    def body(m_ref, whh_ref, bhh_ref, o_ref):
        o_ref[...] = jnp.dot(m_ref[...], whh_ref[...],
                             preferred_element_type=jnp.float32) + bhh_ref[...]

    return pl.pallas_call(
        body,
        grid=(n // blk,),
        in_specs=[
            pl.BlockSpec((blk, d), lambda i: (i, 0)),
            pl.BlockSpec((d, 3 * d), lambda i: (0, 0)),
            pl.BlockSpec((1, 3 * d), lambda i: (0, 0)),
        ],
        out_specs=pl.BlockSpec((blk, 3 * d), lambda i: (i, 0)),
        out_shape=jax.ShapeDtypeStruct((n, 3 * d), jnp.float32),
    )(m, whht, bhh)


def _tc_gru(partials, m, gh_all, wiht, bih, wmt):
    n, d = m.shape
    blk = 1000

    def body(p_ref, m_ref, gh_ref, wih_ref, bih_ref, wm_ref,
             mo_ref, po_ref):
        msgs = p_ref[0] + p_ref[1]
        h = m_ref[...]
        gi = jnp.dot(msgs, wih_ref[...],
                     preferred_element_type=jnp.float32) + bih_ref[...]
        gh = gh_ref[...]
        r = jax.nn.sigmoid(gi[:, :d] + gh[:, :d])
        z = jax.nn.sigmoid(gi[:, d:2 * d] + gh[:, d:2 * d])
        nn = jnp.tanh(gi[:, 2 * d:] + r * gh[:, 2 * d:])
        mnew = (1.0 - z) * nn + z * h
        mo_ref[...] = mnew
        po_ref[...] = jnp.dot(mnew, wm_ref[...],
                              preferred_element_type=jnp.float32)

    return pl.pallas_call(
        body,
        grid=(n // blk,),
        in_specs=[
            pl.BlockSpec((_NC, blk, d), lambda i: (0, i, 0)),
            pl.BlockSpec((blk, d), lambda i: (i, 0)),
            pl.BlockSpec((blk, 3 * d), lambda i: (i, 0)),
            pl.BlockSpec((d, 3 * d), lambda i: (0, 0)),
            pl.BlockSpec((1, 3 * d), lambda i: (0, 0)),
            pl.BlockSpec((d, d), lambda i: (0, 0)),
        ],
        out_specs=[pl.BlockSpec((blk, d), lambda i: (i, 0)),
                   pl.BlockSpec((blk, d), lambda i: (i, 0))],
        out_shape=[jax.ShapeDtypeStruct((n, d), jnp.float32),
                   jax.ShapeDtypeStruct((n, d), jnp.float32)],
    )(partials, m, gh_all, wiht, bih, wmt)


def _tc_readout(m, ids, wat, bat, wpt, bpt, wft, bft):
    n, d = m.shape

    def body(m_ref, ids_ref, wa_ref, ba_ref, wp_ref, bp_ref, wf_ref, bf_ref,
             o_ref):
        mm = m_ref[...]
        attn = jax.nn.sigmoid(
            jnp.dot(mm, wa_ref[...], preferred_element_type=jnp.float32)
            + ba_ref[...])
        pe = jnp.dot(mm, wp_ref[...],
                     preferred_element_type=jnp.float32) + bp_ref[...]
        weighted = attn * pe
        gids = lax.broadcasted_iota(jnp.int32, (_G, n), 0).astype(jnp.float32)
        onehot = jnp.where(gids == ids_ref[...], 1.0, 0.0)
        seg = jnp.dot(onehot, weighted, precision=lax.Precision.HIGHEST,
                      preferred_element_type=jnp.float32)
        o_ref[...] = jnp.dot(seg, wf_ref[...],
                             preferred_element_type=jnp.float32) + bf_ref[...]

    return pl.pallas_call(
        body,
        grid=(1,),
        in_specs=[
            pl.BlockSpec((n, d), lambda i: (0, 0)),
            pl.BlockSpec((1, n), lambda i: (0, 0)),
            pl.BlockSpec((d, 1), lambda i: (0, 0)),
            pl.BlockSpec((1, 1), lambda i: (0, 0)),
            pl.BlockSpec((d, d), lambda i: (0, 0)),
            pl.BlockSpec((1, d), lambda i: (0, 0)),
            pl.BlockSpec((d, 1), lambda i: (0, 0)),
            pl.BlockSpec((1, 1), lambda i: (0, 0)),
        ],
        out_specs=pl.BlockSpec((_G, 1), lambda i: (0, 0)),
        out_shape=jax.ShapeDtypeStruct((_G, 1), jnp.float32),
    )(m, ids, wat, bat, wpt, bpt, wft, bft)


def kernel(node_features, edge_list, node_to_graph_id, num_graphs, W_msg,
           W_ih, b_ih, W_hh, b_hh, W_attn, b_attn, W_proj, b_proj, W_final,
           b_final):
    n, d = node_features.shape
    e = edge_list.shape[0]

    wmt = W_msg.T
    wiht = W_ih.T
    whht = W_hh.T
    bih = b_ih.reshape(1, -1)
    bhh = b_hh.reshape(1, -1)
    wat = W_attn.T
    bat = b_attn.reshape(1, 1)
    wpt = W_proj.T
    bpt = b_proj.reshape(1, -1)
    wft = W_final.T
    bft = b_final.reshape(1, 1)

    dst = edge_list[:, 0]
    src = edge_list[:, 1]
    ept = e // _NW
    src3 = src.reshape(_NW, ept // _CH, _CH)
    dst3 = dst.reshape(_NW, ept // _CH, _CH)
    edge3 = jnp.stack([src3, dst3], axis=2)

    ids = node_to_graph_id.astype(jnp.float32).reshape(1, n)

    spmm = _make_spmm(n, e, d)

    m = node_features
    projs = _tc_proj(m, wmt)
    for _ in range(_T):
        gh_all = _tc_gh(m, whht, bhh)
        partials = spmm(projs, edge3)
        m, projs = _tc_gru(partials, m, gh_all, wiht, bih, wmt)
    return _tc_readout(m, ids, wat, bat, wpt, bpt, wft, bft)


# speedup vs baseline: 1.0269x; 1.0269x over previous
import functools

import jax
import jax.numpy as jnp
from jax import lax
from jax.experimental import pallas as pl
from jax.experimental.pallas import tpu as pltpu
from jax.experimental.pallas import tpu_sc as plsc

_T = 4
_G = 64
_NC = 2
_NS = 16
_NW = _NC * _NS
_CH = 80


@functools.lru_cache(maxsize=None)
def _make_spmm(N, E, D):
    ept = E // _NW
    nch = ept // _CH
    npad = -(-N // 128) * 128
    rpt = npad // _NS
    zfull = rpt // _CH
    ztail = rpt - zfull * _CH
    mesh = plsc.VectorSubcoreMesh(core_axis_name="c", subcore_axis_name="s")

    @functools.partial(
        pl.kernel,
        out_type=jax.ShapeDtypeStruct((_NC, npad, D), jnp.float32),
        mesh=mesh,
        scratch_types=(
            [pltpu.VMEM((2, _CH), jnp.int32) for _ in range(6)]
            + [pltpu.VMEM((_CH, D), jnp.float32) for _ in range(3)]
            + [pltpu.VMEM_SHARED((npad, D), jnp.float32)]
            + [pltpu.SemaphoreType.DMA] * 13
        ),
    )
    def spmm(projs, edge3, out, i0, i1, i2, i3, i4, i5, r0, r1, r2, acc,
             si0, si1, si2, si3, si4, si5, sg0, sg1, sg2, sc0, sc1, sc2, sz):
        c = lax.axis_index("c")
        s = lax.axis_index("s")
        wid = s * _NC + c
        islot = [i0, i1, i2, i3, i4, i5]
        rows = [r0, r1, r2]
        sidx = [si0, si1, si2, si3, si4, si5]
        sg = [sg0, sg1, sg2]
        ssc = [sc0, sc1, sc2]

        nvec = _CH * (D // 16)

        def zfill(i, carry):
            r0[i // (D // 16), pl.ds((i % (D // 16)) * 16, 16)] = (
                jnp.zeros((16,), jnp.float32))
            return carry

        lax.fori_loop(0, nvec, zfill, 0)

        def zcopy(k, carry):
            pltpu.async_copy(r0, acc.at[pl.ds(s * rpt + k * _CH, _CH)], sz)
            return carry

        lax.fori_loop(0, zfull, zcopy, 0)
        if ztail:
            pltpu.async_copy(r0.at[pl.ds(0, ztail)],
                             acc.at[pl.ds(s * rpt + zfull * _CH, ztail)], sz)

        def zwait(k, carry):
            pltpu.make_async_copy(r0, acc.at[pl.ds(0, _CH)], sz).wait()
            return carry

        lax.fori_loop(0, zfull, zwait, 0)
        if ztail:
            pltpu.make_async_copy(r0.at[pl.ds(0, ztail)],
                                  acc.at[pl.ds(0, ztail)], sz).wait()
        plsc.subcore_barrier()

        last = nch - 1

        def fetch_idx(j, q):
            pltpu.async_copy(edge3.at[wid, j], islot[q], sidx[q])

        def wait_idx(q):
            pltpu.make_async_copy(edge3.at[wid, 0], islot[q],
                                  sidx[q]).wait()

        def gather(j, q, g):
            pltpu.async_copy(projs.at[islot[q].at[0]], rows[g], sg[g])

        def wait_gather(b):
            pltpu.make_async_copy(projs.at[islot[0].at[0]], rows[b],
                                  sg[b]).wait()

        def scatter(b, qb):
            pltpu.async_copy(rows[b], acc.at[islot[qb].at[1]], ssc[b],
                             add=True)

        def wait_scatter(b):
            pltpu.make_async_copy(rows[b], acc.at[pl.ds(0, _CH)],
                                  ssc[b]).wait()

        def step(j, jj, k):
            b, g = j % 3, (j + 1) % 3
            q, f, qb = (j + 1) % 6, (j + 4) % 6, j % 6
            jd = j if jj is None else jj + k
            if j >= 2:
                wait_scatter(g)
            if j + 1 <= last:
                wait_idx(q)
                gather(jd + 1, q, g)
            if j + 4 <= last:
                fetch_idx(jd + 4, f)
            wait_gather(b)
            scatter(b, qb)

        for j in range(4):
            fetch_idx(j, j)
        wait_idx(0)
        gather(0, 0, 0)
        for j in range(6):
            step(j, None, None)

        ngrp = (last + 1 - 6 - 5) // 6

        def group(js, carry):
            jj = 6 + 6 * js
            for k in range(6):
                step(6 + k, jj, k)
            return carry

        lax.fori_loop(0, ngrp, group, 0)
        for j in range(6 + 6 * ngrp, last + 1):
            step(j, None, None)
        wait_scatter((last - 1) % 3)
        wait_scatter(last % 3)

        plsc.subcore_barrier()
        pltpu.sync_copy(acc.at[pl.ds(s * rpt, rpt)],
                        out.at[c, pl.ds(s * rpt, rpt)])

    return spmm


def _tc_proj(m, wmt):
    n, d = m.shape
    blk = 1000

    def body(m_ref, w_ref, o_ref):
        o_ref[...] = jnp.dot(m_ref[...], w_ref[...],
                             preferred_element_type=jnp.float32)

    return pl.pallas_call(
        body,
        grid=(n // blk,),
        in_specs=[pl.BlockSpec((blk, d), lambda i: (i, 0)),
                  pl.BlockSpec((d, d), lambda i: (0, 0))],
        out_specs=pl.BlockSpec((blk, d), lambda i: (i, 0)),
        out_shape=jax.ShapeDtypeStruct((n, d), jnp.float32),
    )(m, wmt)


def _tc_gru(partials, m, wiht, bih, whht, bhh, wmt):
    n, d = m.shape
    blk = 1000

    def body(p_ref, m_ref, wih_ref, bih_ref, whh_ref, bhh_ref, wm_ref,
             mo_ref, po_ref):
        msgs = p_ref[0] + p_ref[1]
        h = m_ref[...]
        gi = jnp.dot(msgs, wih_ref[...],
                     preferred_element_type=jnp.float32) + bih_ref[...]
        gh = jnp.dot(h, whh_ref[...],
                     preferred_element_type=jnp.float32) + bhh_ref[...]
        r = jax.nn.sigmoid(gi[:, :d] + gh[:, :d])
        z = jax.nn.sigmoid(gi[:, d:2 * d] + gh[:, d:2 * d])
        nn = jnp.tanh(gi[:, 2 * d:] + r * gh[:, 2 * d:])
        mnew = (1.0 - z) * nn + z * h
        mo_ref[...] = mnew
        po_ref[...] = jnp.dot(mnew, wm_ref[...],
                              preferred_element_type=jnp.float32)

    return pl.pallas_call(
        body,
        grid=(n // blk,),
        in_specs=[
            pl.BlockSpec((_NC, blk, d), lambda i: (0, i, 0)),
            pl.BlockSpec((blk, d), lambda i: (i, 0)),
            pl.BlockSpec((d, 3 * d), lambda i: (0, 0)),
            pl.BlockSpec((1, 3 * d), lambda i: (0, 0)),
            pl.BlockSpec((d, 3 * d), lambda i: (0, 0)),
            pl.BlockSpec((1, 3 * d), lambda i: (0, 0)),
            pl.BlockSpec((d, d), lambda i: (0, 0)),
        ],
        out_specs=[pl.BlockSpec((blk, d), lambda i: (i, 0)),
                   pl.BlockSpec((blk, d), lambda i: (i, 0))],
        out_shape=[jax.ShapeDtypeStruct((n, d), jnp.float32),
                   jax.ShapeDtypeStruct((n, d), jnp.float32)],
    )(partials, m, wiht, bih, whht, bhh, wmt)


def _tc_readout(m, ids, wat, bat, wpt, bpt, wft, bft):
    n, d = m.shape

    def body(m_ref, ids_ref, wa_ref, ba_ref, wp_ref, bp_ref, wf_ref, bf_ref,
             o_ref):
        mm = m_ref[...]
        attn = jax.nn.sigmoid(
            jnp.dot(mm, wa_ref[...], preferred_element_type=jnp.float32)
            + ba_ref[...])
        pe = jnp.dot(mm, wp_ref[...],
                     preferred_element_type=jnp.float32) + bp_ref[...]
        weighted = attn * pe
        gids = lax.broadcasted_iota(jnp.int32, (_G, n), 0).astype(jnp.float32)
        onehot = jnp.where(gids == ids_ref[...], 1.0, 0.0)
        seg = jnp.dot(onehot, weighted, precision=lax.Precision.HIGHEST,
                      preferred_element_type=jnp.float32)
        o_ref[...] = jnp.dot(seg, wf_ref[...],
                             preferred_element_type=jnp.float32) + bf_ref[...]

    return pl.pallas_call(
        body,
        grid=(1,),
        in_specs=[
            pl.BlockSpec((n, d), lambda i: (0, 0)),
            pl.BlockSpec((1, n), lambda i: (0, 0)),
            pl.BlockSpec((d, 1), lambda i: (0, 0)),
            pl.BlockSpec((1, 1), lambda i: (0, 0)),
            pl.BlockSpec((d, d), lambda i: (0, 0)),
            pl.BlockSpec((1, d), lambda i: (0, 0)),
            pl.BlockSpec((d, 1), lambda i: (0, 0)),
            pl.BlockSpec((1, 1), lambda i: (0, 0)),
        ],
        out_specs=pl.BlockSpec((_G, 1), lambda i: (0, 0)),
        out_shape=jax.ShapeDtypeStruct((_G, 1), jnp.float32),
    )(m, ids, wat, bat, wpt, bpt, wft, bft)


def kernel(node_features, edge_list, node_to_graph_id, num_graphs, W_msg,
           W_ih, b_ih, W_hh, b_hh, W_attn, b_attn, W_proj, b_proj, W_final,
           b_final):
    n, d = node_features.shape
    e = edge_list.shape[0]

    wmt = W_msg.T
    wiht = W_ih.T
    whht = W_hh.T
    bih = b_ih.reshape(1, -1)
    bhh = b_hh.reshape(1, -1)
    wat = W_attn.T
    bat = b_attn.reshape(1, 1)
    wpt = W_proj.T
    bpt = b_proj.reshape(1, -1)
    wft = W_final.T
    bft = b_final.reshape(1, 1)

    dst = edge_list[:, 0]
    src = edge_list[:, 1]
    ept = e // _NW
    src3 = src.reshape(_NW, ept // _CH, _CH)
    dst3 = dst.reshape(_NW, ept // _CH, _CH)
    edge3 = jnp.stack([src3, dst3], axis=2)

    ids = node_to_graph_id.astype(jnp.float32).reshape(1, n)

    spmm = _make_spmm(n, e, d)

    m = node_features
    projs = _tc_proj(m, wmt)
    for _ in range(_T):
        partials = spmm(projs, edge3)
        m, projs = _tc_gru(partials, m, wiht, bih, whht, bhh, wmt)
    return _tc_readout(m, ids, wat, bat, wpt, bpt, wft, bft)
